# Initial kernel scaffold; baseline (speedup 1.0000x reference)
import jax
import jax.numpy as jnp
from jax.experimental import pallas as pl


def kernel(x, W_r):
    raise NotImplementedError("write your pallas kernel here")


# speedup vs baseline: 1.3071x; 1.3071x over previous
import jax
import jax.numpy as jnp
from jax.experimental import pallas as pl

_TOKENS = 16384
_D = 2048
_E = 64
_BT = 512


def _router_body(x_ref, wt_ref, probs_ref, gates_ref, idx_ref):
    x = x_ref[...]
    wt = wt_ref[...]
    logits = jax.lax.dot_general(
        x, wt, (((1,), (0,)), ((), ())),
        preferred_element_type=jnp.float32,
        precision=jax.lax.Precision.DEFAULT,
    )
    m = jnp.max(logits, axis=-1, keepdims=True)
    e = jnp.exp(logits - m)
    probs = e / jnp.sum(e, axis=-1, keepdims=True)
    probs_ref[...] = probs

    lane = jax.lax.broadcasted_iota(jnp.int32, probs.shape, 1)
    m1 = jnp.max(probs, axis=-1, keepdims=True)
    i1 = jnp.min(jnp.where(probs == m1, lane, _E), axis=-1, keepdims=True)
    masked = jnp.where(lane == i1, -1.0, probs)
    m2 = jnp.max(masked, axis=-1, keepdims=True)
    i2 = jnp.min(jnp.where(masked == m2, lane, _E), axis=-1, keepdims=True)
    s = m1 + m2
    gates_ref[...] = jnp.concatenate([m1 / s, m2 / s], axis=1)
    idx_ref[...] = jnp.concatenate([i1, i2], axis=1)


def kernel(x, W_r):
    wt = W_r.T
    grid = (_TOKENS // _BT,)
    probs, gates, idx = pl.pallas_call(
        _router_body,
        grid=grid,
        in_specs=[
            pl.BlockSpec((_BT, _D), lambda i: (i, 0)),
            pl.BlockSpec((_D, _E), lambda i: (0, 0)),
        ],
        out_specs=[
            pl.BlockSpec((_BT, _E), lambda i: (i, 0)),
            pl.BlockSpec((_BT, 2), lambda i: (i, 0)),
            pl.BlockSpec((_BT, 2), lambda i: (i, 0)),
        ],
        out_shape=[
            jax.ShapeDtypeStruct((_TOKENS, _E), jnp.float32),
            jax.ShapeDtypeStruct((_TOKENS, 2), jnp.float32),
            jax.ShapeDtypeStruct((_TOKENS, 2), jnp.int32),
        ],
    )(x, wt)
    return (gates, idx, probs)


# speedup vs baseline: 1.3765x; 1.0531x over previous
import jax
import jax.numpy as jnp
from jax.experimental import pallas as pl

_TOKENS = 16384
_D = 2048
_E = 64
_BT = 512


def _router_body(x_ref, wt_ref, probs_ref, gates_ref, idx_ref):
    x = x_ref[...]
    wt = wt_ref[...]
    logits = jax.lax.dot_general(
        x, wt, (((1,), (0,)), ((), ())),
        preferred_element_type=jnp.float32,
        precision=jax.lax.Precision.DEFAULT,
    )
    m = jnp.max(logits, axis=-1, keepdims=True)
    e = jnp.exp(logits - m)
    probs = e / jnp.sum(e, axis=-1, keepdims=True)
    probs_ref[...] = probs

    lane = jax.lax.broadcasted_iota(jnp.int32, probs.shape, 1)
    i1 = jnp.argmax(probs, axis=-1, keepdims=True)
    m1 = jnp.max(probs, axis=-1, keepdims=True)
    masked = jnp.where(lane == i1, -1.0, probs)
    i2 = jnp.argmax(masked, axis=-1, keepdims=True)
    m2 = jnp.max(masked, axis=-1, keepdims=True)
    s = m1 + m2
    gates_ref[...] = jnp.concatenate([m1 / s, m2 / s], axis=1)
    idx_ref[...] = jnp.concatenate([i1, i2], axis=1)


def kernel(x, W_r):
    wt = W_r.T
    grid = (_TOKENS // _BT,)
    probs, gates, idx = pl.pallas_call(
        _router_body,
        grid=grid,
        in_specs=[
            pl.BlockSpec((_BT, _D), lambda i: (i, 0)),
            pl.BlockSpec((_D, _E), lambda i: (0, 0)),
        ],
        out_specs=[
            pl.BlockSpec((_BT, _E), lambda i: (i, 0)),
            pl.BlockSpec((_BT, 2), lambda i: (i, 0)),
            pl.BlockSpec((_BT, 2), lambda i: (i, 0)),
        ],
        out_shape=[
            jax.ShapeDtypeStruct((_TOKENS, _E), jnp.float32),
            jax.ShapeDtypeStruct((_TOKENS, 2), jnp.float32),
            jax.ShapeDtypeStruct((_TOKENS, 2), jnp.int32),
        ],
    )(x, wt)
    return (gates, idx, probs)


# speedup vs baseline: 1.5458x; 1.1230x over previous
import jax
import jax.numpy as jnp
from jax.experimental import pallas as pl

_TOKENS = 16384
_D = 2048
_E = 64
_BT = 1024


def _router_body(x_ref, wt_ref, probs_ref, gates_ref, idx_ref):
    x = x_ref[...]
    wt = wt_ref[...]
    logits = jax.lax.dot_general(
        x, wt, (((1,), (0,)), ((), ())),
        preferred_element_type=jnp.float32,
        precision=jax.lax.Precision.DEFAULT,
    )
    m = jnp.max(logits, axis=-1, keepdims=True)
    e = jnp.exp(logits - m)
    probs = e / jnp.sum(e, axis=-1, keepdims=True)
    probs_ref[...] = probs

    lane = jax.lax.broadcasted_iota(jnp.int32, probs.shape, 1)
    i1 = jnp.argmax(probs, axis=-1, keepdims=True)
    m1 = jnp.max(probs, axis=-1, keepdims=True)
    masked = jnp.where(lane == i1, -1.0, probs)
    i2 = jnp.argmax(masked, axis=-1, keepdims=True)
    m2 = jnp.max(masked, axis=-1, keepdims=True)
    s = m1 + m2
    gates_ref[...] = jnp.concatenate([m1 / s, m2 / s], axis=1)
    idx_ref[...] = jnp.concatenate([i1, i2], axis=1)


def kernel(x, W_r):
    wt = W_r.T
    grid = (_TOKENS // _BT,)
    probs, gates, idx = pl.pallas_call(
        _router_body,
        grid=grid,
        in_specs=[
            pl.BlockSpec((_BT, _D), lambda i: (i, 0)),
            pl.BlockSpec((_D, _E), lambda i: (0, 0)),
        ],
        out_specs=[
            pl.BlockSpec((_BT, _E), lambda i: (i, 0)),
            pl.BlockSpec((_BT, 2), lambda i: (i, 0)),
            pl.BlockSpec((_BT, 2), lambda i: (i, 0)),
        ],
        out_shape=[
            jax.ShapeDtypeStruct((_TOKENS, _E), jnp.float32),
            jax.ShapeDtypeStruct((_TOKENS, 2), jnp.float32),
            jax.ShapeDtypeStruct((_TOKENS, 2), jnp.int32),
        ],
    )(x, wt)
    return (gates, idx, probs)


# speedup vs baseline: 1.5871x; 1.0267x over previous
import jax
import jax.numpy as jnp
from jax.experimental import pallas as pl

_TOKENS = 16384
_D = 2048
_E = 64
_BT = 2048


def _router_body(x_ref, wt_ref, probs_ref, gates_ref, idx_ref):
    x = x_ref[...]
    wt = wt_ref[...]
    logits = jax.lax.dot_general(
        x, wt, (((1,), (0,)), ((), ())),
        preferred_element_type=jnp.float32,
        precision=jax.lax.Precision.DEFAULT,
    )
    m = jnp.max(logits, axis=-1, keepdims=True)
    e = jnp.exp(logits - m)
    probs = e / jnp.sum(e, axis=-1, keepdims=True)
    probs_ref[...] = probs

    lane = jax.lax.broadcasted_iota(jnp.int32, probs.shape, 1)
    i1 = jnp.argmax(probs, axis=-1, keepdims=True)
    m1 = jnp.max(probs, axis=-1, keepdims=True)
    masked = jnp.where(lane == i1, -1.0, probs)
    i2 = jnp.argmax(masked, axis=-1, keepdims=True)
    m2 = jnp.max(masked, axis=-1, keepdims=True)
    s = m1 + m2
    gates_ref[...] = jnp.concatenate([m1 / s, m2 / s], axis=1)
    idx_ref[...] = jnp.concatenate([i1, i2], axis=1)


def kernel(x, W_r):
    wt = W_r.T
    grid = (_TOKENS // _BT,)
    probs, gates, idx = pl.pallas_call(
        _router_body,
        grid=grid,
        in_specs=[
            pl.BlockSpec((_BT, _D), lambda i: (i, 0)),
            pl.BlockSpec((_D, _E), lambda i: (0, 0)),
        ],
        out_specs=[
            pl.BlockSpec((_BT, _E), lambda i: (i, 0)),
            pl.BlockSpec((_BT, 2), lambda i: (i, 0)),
            pl.BlockSpec((_BT, 2), lambda i: (i, 0)),
        ],
        out_shape=[
            jax.ShapeDtypeStruct((_TOKENS, _E), jnp.float32),
            jax.ShapeDtypeStruct((_TOKENS, 2), jnp.float32),
            jax.ShapeDtypeStruct((_TOKENS, 2), jnp.int32),
        ],
    )(x, wt)
    return (gates, idx, probs)
